# baseline (speedup 1.0000x reference)
import functools

import jax
import jax.numpy as jnp
from jax import lax
from jax.experimental import pallas as pl
from jax.experimental.pallas import tpu as pltpu
from jax.experimental.pallas import tpu_sc as plsc

_NC = 2
_NS = 16
_NW = _NC * _NS
_K = 8
_NBUF = 3


def _make(nb, d):
    half = d // 2
    bpw = nb // _NW
    nch = bpw // _K
    nst = 2 * nch
    mesh = plsc.VectorSubcoreMesh(core_axis_name="c", subcore_axis_name="s")

    @functools.partial(
        pl.kernel,
        out_type=jax.ShapeDtypeStruct((nb, d), jnp.float32),
        mesh=mesh,
        scratch_types=(
            [pltpu.VMEM((bpw,), jnp.int32)]
            + [pltpu.VMEM((_K, half), jnp.float32)] * _NBUF
            + [pltpu.SemaphoreType.DMA] * (2 * _NBUF)
        ),
    )
    def emb(idx_hbm, table_hbm, out_hbm, idx_v, *rest):
        bufs = rest[:_NBUF]
        gsem = rest[_NBUF:2 * _NBUF]
        wsem = rest[2 * _NBUF:]
        wid = lax.axis_index("s") * _NC + lax.axis_index("c")
        base = wid * bpw
        pltpu.sync_copy(idx_hbm.at[pl.ds(base, bpw)], idx_v)

        def coords(s):
            return (s % nch) * _K, (s // nch) * half

        def start_g(s, buf, sem):
            off, col = coords(s)
            return pltpu.async_copy(
                table_hbm.at[idx_v.at[pl.ds(off, _K)], pl.ds(col, half)],
                buf, sem)

        def start_w(s, buf, sem):
            off, col = coords(s)
            row = pl.multiple_of(base, 8) + off
            return pltpu.async_copy(
                buf, out_hbm.at[pl.ds(row, _K), pl.ds(col, half)], sem)

        def wait_g(buf, sem):
            pltpu.make_async_copy(
                table_hbm.at[pl.ds(0, _K), pl.ds(0, half)], buf, sem).wait()

        def wait_w(buf, sem):
            pltpu.make_async_copy(
                buf, out_hbm.at[pl.ds(0, _K), pl.ds(0, half)], sem).wait()

        start_g(0, bufs[0], gsem[0])
        for t in range(nst):
            b = t % _NBUF
            bn = (t + 1) % _NBUF
            if t + 1 < nst:
                if t - 2 >= 0:
                    wait_w(bufs[bn], wsem[bn])
                start_g(t + 1, bufs[bn], gsem[bn])
            wait_g(bufs[b], gsem[b])
            start_w(t, bufs[b], wsem[b])
        for t in range(nst - _NBUF, nst):
            b = t % _NBUF
            wait_w(bufs[b], wsem[b])

    return emb


def kernel(idx, table):
    b, t = idx.shape
    v, d = table.shape
    nb = b * t
    idx_flat = idx.reshape(nb).astype(jnp.int32)
    out = _make(nb, d)(idx_flat, table)
    return out.reshape(b, t, d)


# speedup vs baseline: 1.0041x; 1.0041x over previous
import functools

import jax
import jax.numpy as jnp
from jax import lax
from jax.experimental import pallas as pl
from jax.experimental.pallas import tpu as pltpu
from jax.experimental.pallas import tpu_sc as plsc

_NC = 2
_NS = 16
_NW = _NC * _NS
_K = 8
_NBUF = 3


def _make(nb, d):
    half = d // 2
    bpw = nb // _NW
    nst = 2 * (bpw // _K)
    mesh = plsc.VectorSubcoreMesh(core_axis_name="c", subcore_axis_name="s")

    @functools.partial(
        pl.kernel,
        out_type=jax.ShapeDtypeStruct((nb, d), jnp.float32),
        mesh=mesh,
        scratch_types=(
            [pltpu.VMEM((bpw,), jnp.int32)]
            + [pltpu.VMEM((_K, half), jnp.float32)] * _NBUF
            + [pltpu.SemaphoreType.DMA] * (2 * _NBUF)
        ),
    )
    def emb(idx_hbm, table_hbm, out_hbm, idx_v, *rest):
        bufs = rest[:_NBUF]
        gsem = rest[_NBUF:2 * _NBUF]
        wsem = rest[2 * _NBUF:]
        wid = lax.axis_index("s") * _NC + lax.axis_index("c")
        base = wid * bpw
        pltpu.sync_copy(idx_hbm.at[pl.ds(base, bpw)], idx_v)

        def coords(s):
            return (s // 2) * _K, (s % 2) * half

        def start_g(s, buf, sem):
            off, col = coords(s)
            return pltpu.async_copy(
                table_hbm.at[idx_v.at[pl.ds(off, _K)], pl.ds(col, half)],
                buf, sem)

        def start_w(s, buf, sem):
            off, col = coords(s)
            row = pl.multiple_of(base, 8) + off
            return pltpu.async_copy(
                buf, out_hbm.at[pl.ds(row, _K), pl.ds(col, half)], sem)

        def wait_g(buf, sem):
            pltpu.make_async_copy(
                table_hbm.at[pl.ds(0, _K), pl.ds(0, half)], buf, sem).wait()

        def wait_w(buf, sem):
            pltpu.make_async_copy(
                buf, out_hbm.at[pl.ds(0, _K), pl.ds(0, half)], sem).wait()

        start_g(0, bufs[0], gsem[0])
        for t in range(nst):
            b = t % _NBUF
            bn = (t + 1) % _NBUF
            if t + 1 < nst:
                if t - 2 >= 0:
                    wait_w(bufs[bn], wsem[bn])
                start_g(t + 1, bufs[bn], gsem[bn])
            wait_g(bufs[b], gsem[b])
            start_w(t, bufs[b], wsem[b])
        for t in range(nst - _NBUF, nst):
            b = t % _NBUF
            wait_w(bufs[b], wsem[b])

    return emb


def kernel(idx, table):
    b, t = idx.shape
    v, d = table.shape
    nb = b * t
    idx_flat = idx.reshape(nb).astype(jnp.int32)
    out = _make(nb, d)(idx_flat, table)
    return out.reshape(b, t, d)


# speedup vs baseline: 1.0249x; 1.0208x over previous
import functools

import jax
import jax.numpy as jnp
from jax import lax
from jax.experimental import pallas as pl
from jax.experimental.pallas import tpu as pltpu
from jax.experimental.pallas import tpu_sc as plsc

_NC = 2
_NS = 16
_NW = _NC * _NS
_K = 8
_NBUF = 2
_BW = 3584


def _make(nb, d):
    half = d // 2
    rest_w = half - _BW
    bpw = nb // _NW
    nst = 2 * (bpw // _K)
    mesh = plsc.VectorSubcoreMesh(core_axis_name="c", subcore_axis_name="s")

    @functools.partial(
        pl.kernel,
        out_type=jax.ShapeDtypeStruct((nb, d), jnp.float32),
        mesh=mesh,
        scratch_types=(
            [pltpu.VMEM((bpw,), jnp.int32)]
            + [pltpu.VMEM((_K, half), jnp.float32)] * _NBUF
            + [pltpu.VMEM_SHARED((_NS * _K, _BW), jnp.float32)] * _NBUF
            + [pltpu.SemaphoreType.DMA] * (4 * _NBUF)
        ),
    )
    def emb(idx_hbm, table_hbm, out_hbm, idx_v, *rest):
        tbufs = rest[:_NBUF]
        shared = rest[_NBUF:2 * _NBUF]
        gsem = rest[2 * _NBUF:3 * _NBUF]
        csem = rest[3 * _NBUF:4 * _NBUF]
        wsem = rest[4 * _NBUF:5 * _NBUF]
        dsem = rest[5 * _NBUF:]
        wid = lax.axis_index("s") * _NC + lax.axis_index("c")
        base = wid * bpw
        trow = pl.multiple_of(lax.axis_index("s") * _K, 8)
        sbufs = [sb.at[pl.ds(trow, _K), :] for sb in shared]
        pltpu.sync_copy(idx_hbm.at[pl.ds(base, bpw)], idx_v)

        def coords(s):
            return (s // 2) * _K, (s % 2) * half

        def start_g(s, b):
            off, col = coords(s)
            return pltpu.async_copy(
                table_hbm.at[idx_v.at[pl.ds(off, _K)], pl.ds(col, half)],
                tbufs[b], gsem[b])

        def start_c(b):
            return pltpu.async_copy(
                tbufs[b].at[:, pl.ds(0, _BW)], sbufs[b], csem[b])

        def start_d(s, b):
            off, col = coords(s)
            row = pl.multiple_of(base, 8) + off
            return pltpu.async_copy(
                tbufs[b].at[:, pl.ds(_BW, rest_w)],
                out_hbm.at[pl.ds(row, _K), pl.ds(col + _BW, rest_w)],
                dsem[b])

        def start_w(s, b):
            off, col = coords(s)
            row = pl.multiple_of(base, 8) + off
            return pltpu.async_copy(
                sbufs[b], out_hbm.at[pl.ds(row, _K), pl.ds(col, _BW)],
                wsem[b])

        def wait_g(b):
            pltpu.make_async_copy(
                table_hbm.at[pl.ds(0, _K), pl.ds(0, half)], tbufs[b],
                gsem[b]).wait()

        def wait_c(b):
            pltpu.make_async_copy(
                tbufs[b].at[:, pl.ds(0, _BW)], sbufs[b], csem[b]).wait()

        def wait_d(b):
            pltpu.make_async_copy(
                tbufs[b].at[:, pl.ds(_BW, rest_w)],
                out_hbm.at[pl.ds(0, _K), pl.ds(0, rest_w)], dsem[b]).wait()

        def wait_w(b):
            pltpu.make_async_copy(
                sbufs[b], out_hbm.at[pl.ds(0, _K), pl.ds(0, _BW)],
                wsem[b]).wait()

        start_g(0, 0)
        for t in range(nst):
            b = t % _NBUF
            bp = (t - 1) % _NBUF
            bn = (t + 1) % _NBUF
            if t >= 1:
                wait_c(bp)
                wait_d(bp)
                start_w(t - 1, bp)
            if t + 1 < nst:
                start_g(t + 1, bn)
            wait_g(b)
            if t - _NBUF >= 0:
                wait_w(b)
            start_c(b)
            start_d(t, b)
        bl = (nst - 1) % _NBUF
        wait_c(bl)
        wait_d(bl)
        start_w(nst - 1, bl)
        for t in range(nst - _NBUF, nst):
            wait_w(t % _NBUF)

    return emb


def kernel(idx, table):
    b, t = idx.shape
    v, d = table.shape
    nb = b * t
    idx_flat = idx.reshape(nb).astype(jnp.int32)
    out = _make(nb, d)(idx_flat, table)
    return out.reshape(b, t, d)


# speedup vs baseline: 1.0287x; 1.0037x over previous
import functools

import jax
import jax.numpy as jnp
from jax import lax
from jax.experimental import pallas as pl
from jax.experimental.pallas import tpu as pltpu
from jax.experimental.pallas import tpu_sc as plsc

_NC = 2
_NS = 16
_NW = _NC * _NS
_K = 8
_NBUF = 2
_BW = 3968


def _make(nb, d):
    half = d // 2
    rest_w = half - _BW
    bpw = nb // _NW
    nst = 2 * (bpw // _K)
    mesh = plsc.VectorSubcoreMesh(core_axis_name="c", subcore_axis_name="s")

    @functools.partial(
        pl.kernel,
        out_type=jax.ShapeDtypeStruct((nb, d), jnp.float32),
        mesh=mesh,
        scratch_types=(
            [pltpu.VMEM((bpw,), jnp.int32)]
            + [pltpu.VMEM((_K, half), jnp.float32)] * _NBUF
            + [pltpu.VMEM_SHARED((_NS * _K, _BW), jnp.float32)] * _NBUF
            + [pltpu.SemaphoreType.DMA] * (4 * _NBUF)
        ),
    )
    def emb(idx_hbm, table_hbm, out_hbm, idx_v, *rest):
        tbufs = rest[:_NBUF]
        shared = rest[_NBUF:2 * _NBUF]
        gsem = rest[2 * _NBUF:3 * _NBUF]
        csem = rest[3 * _NBUF:4 * _NBUF]
        wsem = rest[4 * _NBUF:5 * _NBUF]
        dsem = rest[5 * _NBUF:]
        wid = lax.axis_index("s") * _NC + lax.axis_index("c")
        base = wid * bpw
        trow = pl.multiple_of(lax.axis_index("s") * _K, 8)
        sbufs = [sb.at[pl.ds(trow, _K), :] for sb in shared]
        pltpu.sync_copy(idx_hbm.at[pl.ds(base, bpw)], idx_v)

        def coords(s):
            return (s // 2) * _K, (s % 2) * half

        def start_g(s, b):
            off, col = coords(s)
            return pltpu.async_copy(
                table_hbm.at[idx_v.at[pl.ds(off, _K)], pl.ds(col, half)],
                tbufs[b], gsem[b])

        def start_c(b):
            return pltpu.async_copy(
                tbufs[b].at[:, pl.ds(0, _BW)], sbufs[b], csem[b])

        def start_d(s, b):
            off, col = coords(s)
            row = pl.multiple_of(base, 8) + off
            return pltpu.async_copy(
                tbufs[b].at[:, pl.ds(_BW, rest_w)],
                out_hbm.at[pl.ds(row, _K), pl.ds(col + _BW, rest_w)],
                dsem[b])

        def start_w(s, b):
            off, col = coords(s)
            row = pl.multiple_of(base, 8) + off
            return pltpu.async_copy(
                sbufs[b], out_hbm.at[pl.ds(row, _K), pl.ds(col, _BW)],
                wsem[b])

        def wait_g(b):
            pltpu.make_async_copy(
                table_hbm.at[pl.ds(0, _K), pl.ds(0, half)], tbufs[b],
                gsem[b]).wait()

        def wait_c(b):
            pltpu.make_async_copy(
                tbufs[b].at[:, pl.ds(0, _BW)], sbufs[b], csem[b]).wait()

        def wait_d(b):
            pltpu.make_async_copy(
                tbufs[b].at[:, pl.ds(_BW, rest_w)],
                out_hbm.at[pl.ds(0, _K), pl.ds(0, rest_w)], dsem[b]).wait()

        def wait_w(b):
            pltpu.make_async_copy(
                sbufs[b], out_hbm.at[pl.ds(0, _K), pl.ds(0, _BW)],
                wsem[b]).wait()

        start_g(0, 0)
        for t in range(nst):
            b = t % _NBUF
            bp = (t - 1) % _NBUF
            bn = (t + 1) % _NBUF
            if t >= 1:
                wait_c(bp)
                wait_d(bp)
                start_w(t - 1, bp)
            if t + 1 < nst:
                start_g(t + 1, bn)
            wait_g(b)
            if t - _NBUF >= 0:
                wait_w(b)
            start_c(b)
            start_d(t, b)
        bl = (nst - 1) % _NBUF
        wait_c(bl)
        wait_d(bl)
        start_w(nst - 1, bl)
        for t in range(nst - _NBUF, nst):
            wait_w(t % _NBUF)

    return emb


def kernel(idx, table):
    b, t = idx.shape
    v, d = table.shape
    nb = b * t
    idx_flat = idx.reshape(nb).astype(jnp.int32)
    out = _make(nb, d)(idx_flat, table)
    return out.reshape(b, t, d)
